# Initial kernel scaffold; baseline (speedup 1.0000x reference)
import jax
import jax.numpy as jnp
from jax.experimental import pallas as pl


def kernel(node_features, edge_features, W1, W2, from0, to0_pos, eidx0, nodes1, from1, to1_pos, eidx1, batch_pos):
    raise NotImplementedError("write your pallas kernel here")


# speedup vs baseline: 3.4871x; 3.4871x over previous
import functools

import jax
import jax.numpy as jnp
from jax import lax
from jax.experimental import pallas as pl
from jax.experimental.pallas import tpu as pltpu
from jax.experimental.pallas import tpu_sc as plsc

BLK = 128


def _sc_body(nblk0, nblk1,
             nf0, nf1, ef, from0, to0, eid0, n1h, from1, to1, eid1, bph,
             z64, z16,
             x1n_o, s1_o, a2n_o, x1e_o, a2e_o,
             acc1n, acc1e, acc2n, acc2e,
             idx_f, idx_t, idx_e, bpv, idxb, rows_n, rows_e, sem):
    cid = lax.axis_index("c")
    sid = lax.axis_index("s")
    n1 = acc1n.shape[0]
    b = acc2n.shape[0]

    rows1 = n1 // 16
    rows2 = b // 16

    pltpu.sync_copy(z64, acc1n.at[pl.ds(sid * rows1, rows1)])
    pltpu.sync_copy(z16, acc1e.at[pl.ds(sid * rows1, rows1)])
    pltpu.sync_copy(z64.at[pl.ds(0, rows2)], acc2n.at[pl.ds(sid * rows2, rows2)])
    pltpu.sync_copy(z16.at[pl.ds(0, rows2)], acc2e.at[pl.ds(sid * rows2, rows2)])
    plsc.subcore_barrier()

    def agg(from_h, to_h, eid_h, nf_h, acc_n, acc_e, nblk, with_e):
        def blk(i, carry):
            base = sid * (nblk * BLK) + i * BLK
            pltpu.sync_copy(from_h.at[pl.ds(base, BLK)], idx_f)
            pltpu.sync_copy(to_h.at[pl.ds(base, BLK)], idx_t)
            pltpu.async_copy(nf_h.at[idx_f], rows_n, sem).wait()
            pltpu.sync_copy(rows_n, acc_n.at[idx_t], add=True)
            if with_e:
                pltpu.sync_copy(eid_h.at[pl.ds(base, BLK)], idx_e)
                pltpu.async_copy(ef.at[idx_e], rows_e, sem).wait()
                pltpu.sync_copy(rows_e, acc_e.at[idx_t], add=True)
            return carry
        lax.fori_loop(0, nblk, blk, 0)

    @pl.when(cid == 0)
    def _():
        agg(from0, to0, eid0, nf0, acc1n, acc1e, nblk0, False)
        agg(from1, to1, eid1, nf0, acc2n, acc2e, nblk1, False)

    @pl.when(cid == 1)
    def _():
        agg(from0, to0, eid0, nf1, acc1n, acc1e, nblk0, True)
        agg(from1, to1, eid1, nf1, acc2n, acc2e, nblk1, True)

    plsc.subcore_barrier()

    brows = b // 16
    obase = cid * b + sid * brows
    pltpu.sync_copy(bph.at[pl.ds(sid * brows, brows)], bpv)

    pltpu.async_copy(acc1n.at[bpv], rows_n, sem).wait()
    pltpu.sync_copy(rows_n, x1n_o.at[pl.ds(obase, brows)])

    pltpu.async_copy(n1h.at[bpv], idxb, sem).wait()

    @pl.when(cid == 0)
    def _():
        pltpu.async_copy(nf0.at[idxb], rows_n, sem).wait()

    @pl.when(cid == 1)
    def _():
        pltpu.async_copy(nf1.at[idxb], rows_n, sem).wait()

    pltpu.sync_copy(rows_n, s1_o.at[pl.ds(obase, brows)])
    pltpu.sync_copy(acc2n.at[pl.ds(sid * brows, brows)],
                    a2n_o.at[pl.ds(obase, brows)])

    @pl.when(cid == 1)
    def _():
        pltpu.async_copy(acc1e.at[bpv], rows_e, sem).wait()
        pltpu.sync_copy(rows_e, x1e_o.at[pl.ds(sid * brows, brows)])
        pltpu.sync_copy(acc2e.at[pl.ds(sid * brows, brows)],
                        a2e_o.at[pl.ds(sid * brows, brows)])


def _tc_body(s1, x1n, a2n, x1e, a2e, w1, w2, out):
    def dot(x, w):
        return lax.dot_general(x, w, (((1,), (1,)), ((), ())),
                               preferred_element_type=jnp.float32)
    b = out.shape[0]
    h1 = jax.nn.relu(
        dot(s1[0:b, :], w1[:, 0:64]) + dot(s1[b:2 * b, :], w1[:, 64:128])
        + dot(x1n[0:b, :], w1[:, 128:192]) + dot(x1n[b:2 * b, :], w1[:, 192:256])
        + dot(x1e[...], w1[:, 256:272]))
    h2 = jax.nn.relu(
        dot(h1, w2[:, 0:128])
        + dot(a2n[0:b, :], w2[:, 128:192]) + dot(a2n[b:2 * b, :], w2[:, 192:256])
        + dot(a2e[...], w2[:, 256:272]))
    out[...] = h2


def kernel(node_features, edge_features, W1, W2,
           from0, to0_pos, eidx0, nodes1, from1, to1_pos, eidx1, batch_pos):
    n_nodes, d = node_features.shape
    n1 = nodes1.shape[0]
    b = batch_pos.shape[0]
    e0 = from0.shape[0]
    e1 = from1.shape[0]
    dh = d // 2
    nblk0 = e0 // 16 // BLK
    nblk1 = e1 // 16 // BLK

    nf0 = node_features[:, :dh]
    nf1 = node_features[:, dh:]
    z64 = jnp.zeros((n1 // 16, dh), jnp.float32)
    z16 = jnp.zeros((n1 // 16, 16), jnp.float32)

    mesh = plsc.VectorSubcoreMesh(core_axis_name="c", subcore_axis_name="s")
    sc = pl.kernel(
        functools.partial(_sc_body, nblk0, nblk1),
        out_type=[
            jax.ShapeDtypeStruct((2 * b, dh), jnp.float32),
            jax.ShapeDtypeStruct((2 * b, dh), jnp.float32),
            jax.ShapeDtypeStruct((2 * b, dh), jnp.float32),
            jax.ShapeDtypeStruct((b, 16), jnp.float32),
            jax.ShapeDtypeStruct((b, 16), jnp.float32),
        ],
        mesh=mesh,
        scratch_types=[
            pltpu.VMEM_SHARED((n1, dh), jnp.float32),
            pltpu.VMEM_SHARED((n1, 16), jnp.float32),
            pltpu.VMEM_SHARED((b, dh), jnp.float32),
            pltpu.VMEM_SHARED((b, 16), jnp.float32),
            pltpu.VMEM((BLK,), jnp.int32),
            pltpu.VMEM((BLK,), jnp.int32),
            pltpu.VMEM((BLK,), jnp.int32),
            pltpu.VMEM((b // 16,), jnp.int32),
            pltpu.VMEM((b // 16,), jnp.int32),
            pltpu.VMEM((BLK, dh), jnp.float32),
            pltpu.VMEM((BLK, 16), jnp.float32),
            pltpu.SemaphoreType.DMA,
        ],
        compiler_params=pltpu.CompilerParams(use_tc_tiling_on_sc=False),
    )
    x1n, s1, a2n, x1e, a2e = sc(nf0, nf1, edge_features,
                                from0, to0_pos, eidx0, nodes1,
                                from1, to1_pos, eidx1, batch_pos, z64, z16)

    return pl.pallas_call(
        _tc_body,
        out_shape=jax.ShapeDtypeStruct((b, d), jnp.float32),
    )(s1, x1n, a2n, x1e, a2e, W1, W2)


# speedup vs baseline: 6.0950x; 1.7478x over previous
import functools

import jax
import jax.numpy as jnp
from jax import lax
from jax.experimental import pallas as pl
from jax.experimental.pallas import tpu as pltpu
from jax.experimental.pallas import tpu_sc as plsc

BLK = 128


def _sc_body(chunk0, chunk1,
             nf0, nf1, ef, from0, to0, eid0, n1h, from1, to1, eid1, bph,
             z64, z16, zi,
             x1n_o, s1_o, a2n_o, x1e_o, a2e_o,
             acc1n, acc1e, acc2n, acc2e,
             idx_f, idx_t, idx_e, bpv, idxb, rows_n, rows_e,
             mark, bpall, to_buf, kept,
             sem, sem2, sem3):
    cid = lax.axis_index("c")
    sid = lax.axis_index("s")
    n1 = mark.shape[0]
    b = bpall.shape[0]
    e0 = chunk0 * 16
    e1 = chunk1 * 16

    rows1 = n1 // 16
    rows2 = b // 16

    pltpu.sync_copy(z64, acc1n.at[pl.ds(sid * rows1, rows1)])
    pltpu.sync_copy(z16, acc1e.at[pl.ds(sid * rows1, rows1)])
    pltpu.sync_copy(z64.at[pl.ds(0, rows2)], acc2n.at[pl.ds(sid * rows2, rows2)])
    pltpu.sync_copy(z16.at[pl.ds(0, rows2)], acc2e.at[pl.ds(sid * rows2, rows2)])

    pltpu.sync_copy(zi, mark)
    pltpu.sync_copy(bph, bpall)
    ones = jnp.ones((16,), jnp.int32)

    def mark_blk(j, c):
        plsc.store_scatter(mark, [bpall[pl.ds(j * 16, 16)]], ones)
        return c
    lax.fori_loop(0, b // 16, mark_blk, 0)

    plsc.subcore_barrier()
    lanes = lax.iota(jnp.int32, 16)

    def agg(pos_ref, nblk, from_h, to_h, eid_h, nf_h, acc_n, acc_e, with_e):
        def blk(i, c):
            pslice = pos_ref.at[pl.ds(i * BLK, BLK)]
            cp_t = pltpu.async_copy(to_h.at[pslice], idx_t, sem)
            cp_f = pltpu.async_copy(from_h.at[pslice], idx_f, sem2)
            if with_e:
                cp_e = pltpu.async_copy(eid_h.at[pslice], idx_e, sem3)
            cp_f.wait()
            cp_rn = pltpu.async_copy(nf_h.at[idx_f], rows_n, sem2)
            if with_e:
                cp_e.wait()
                cp_re = pltpu.async_copy(ef.at[idx_e], rows_e, sem3)
            cp_t.wait()
            cp_rn.wait()
            pltpu.sync_copy(rows_n, acc_n.at[idx_t], add=True)
            if with_e:
                cp_re.wait()
                pltpu.sync_copy(rows_e, acc_e.at[idx_t], add=True)
            return c
        lax.fori_loop(0, nblk, blk, 0)

    sub = to_buf.shape[0]

    def round_body(r, c):
        rbase = sid * chunk0 + r * sub
        pltpu.sync_copy(to0.at[pl.ds(rbase, sub)], to_buf)

        def filt_blk(i, cnt):
            t16 = to_buf[pl.ds(i * 16, 16)]
            live = plsc.load_gather(mark, [t16]) > 0
            pos16 = rbase + i * 16 + lanes
            plsc.store_compressed(kept.at[pl.ds(cnt, 16)], pos16, mask=live)
            return cnt + jnp.max(plsc.all_reduce_population_count(live))
        cnt = lax.fori_loop(0, sub // 16, filt_blk, jnp.int32(0))

        for j in range(BLK // 16):
            kept[pl.ds(cnt + j * 16, 16)] = jnp.full((16,), e0, jnp.int32)

        nblk0 = (cnt + BLK - 1) // BLK

        @pl.when(cid == 0)
        def _():
            agg(kept, nblk0, from0, to0, eid0, nf0, acc1n, acc1e, False)

        @pl.when(cid == 1)
        def _():
            agg(kept, nblk0, from0, to0, eid0, nf1, acc1n, acc1e, True)

        return c
    lax.fori_loop(0, chunk0 // sub, round_body, 0)

    def fill_blk(i, c):
        kept[pl.ds(i * 16, 16)] = sid * chunk1 + i * 16 + lanes
        return c
    lax.fori_loop(0, chunk1 // 16, fill_blk, 0)

    @pl.when(cid == 0)
    def _():
        agg(kept, chunk1 // BLK, from1, to1, eid1, nf0, acc2n, acc2e, False)

    @pl.when(cid == 1)
    def _():
        agg(kept, chunk1 // BLK, from1, to1, eid1, nf1, acc2n, acc2e, True)

    plsc.subcore_barrier()

    brows = b // 16
    obase = cid * b + sid * brows
    pltpu.sync_copy(bph.at[pl.ds(sid * brows, brows)], bpv)

    pltpu.async_copy(acc1n.at[bpv], rows_n, sem).wait()
    pltpu.sync_copy(rows_n, x1n_o.at[pl.ds(obase, brows)])

    pltpu.async_copy(n1h.at[bpv], idxb, sem).wait()

    @pl.when(cid == 0)
    def _():
        pltpu.async_copy(nf0.at[idxb], rows_n, sem).wait()

    @pl.when(cid == 1)
    def _():
        pltpu.async_copy(nf1.at[idxb], rows_n, sem).wait()

    pltpu.sync_copy(rows_n, s1_o.at[pl.ds(obase, brows)])
    pltpu.sync_copy(acc2n.at[pl.ds(sid * brows, brows)],
                    a2n_o.at[pl.ds(obase, brows)])

    @pl.when(cid == 1)
    def _():
        pltpu.async_copy(acc1e.at[bpv], rows_e, sem).wait()
        pltpu.sync_copy(rows_e, x1e_o.at[pl.ds(sid * brows, brows)])
        pltpu.sync_copy(acc2e.at[pl.ds(sid * brows, brows)],
                        a2e_o.at[pl.ds(sid * brows, brows)])


def _tc_body(s1, x1n, a2n, x1e, a2e, w1, w2, out):
    def dot(x, w):
        return lax.dot_general(x, w, (((1,), (1,)), ((), ())),
                               preferred_element_type=jnp.float32)
    b = out.shape[0]
    h1 = jax.nn.relu(
        dot(s1[0:b, :], w1[:, 0:64]) + dot(s1[b:2 * b, :], w1[:, 64:128])
        + dot(x1n[0:b, :], w1[:, 128:192]) + dot(x1n[b:2 * b, :], w1[:, 192:256])
        + dot(x1e[...], w1[:, 256:272]))
    h2 = jax.nn.relu(
        dot(h1, w2[:, 0:128])
        + dot(a2n[0:b, :], w2[:, 128:192]) + dot(a2n[b:2 * b, :], w2[:, 192:256])
        + dot(a2e[...], w2[:, 256:272]))
    out[...] = h2


def kernel(node_features, edge_features, W1, W2,
           from0, to0_pos, eidx0, nodes1, from1, to1_pos, eidx1, batch_pos):
    n_nodes, d = node_features.shape
    n1 = nodes1.shape[0]
    b = batch_pos.shape[0]
    e0 = from0.shape[0]
    e1 = from1.shape[0]
    dh = d // 2
    chunk0 = e0 // 16
    chunk1 = e1 // 16

    nf0 = node_features[:, :dh]
    nf1 = node_features[:, dh:]
    z64 = jnp.zeros((n1 // 16, dh), jnp.float32)
    z16 = jnp.zeros((n1 // 16, 16), jnp.float32)
    zi = jnp.zeros((n1,), jnp.int32)

    i32 = jnp.int32
    from0x = jnp.concatenate([from0, jnp.zeros((BLK,), i32)])
    to0x = jnp.concatenate([to0_pos, jnp.full((BLK,), n1, i32)])
    eid0x = jnp.concatenate([eidx0, jnp.zeros((BLK,), i32)])

    mesh = plsc.VectorSubcoreMesh(core_axis_name="c", subcore_axis_name="s")
    sc = pl.kernel(
        functools.partial(_sc_body, chunk0, chunk1),
        out_type=[
            jax.ShapeDtypeStruct((2 * b, dh), jnp.float32),
            jax.ShapeDtypeStruct((2 * b, dh), jnp.float32),
            jax.ShapeDtypeStruct((2 * b, dh), jnp.float32),
            jax.ShapeDtypeStruct((b, 16), jnp.float32),
            jax.ShapeDtypeStruct((b, 16), jnp.float32),
        ],
        mesh=mesh,
        scratch_types=[
            pltpu.VMEM_SHARED((n1 + 16, dh), jnp.float32),
            pltpu.VMEM_SHARED((n1 + 16, 16), jnp.float32),
            pltpu.VMEM_SHARED((b + 16, dh), jnp.float32),
            pltpu.VMEM_SHARED((b + 16, 16), jnp.float32),
            pltpu.VMEM((BLK,), jnp.int32),
            pltpu.VMEM((BLK,), jnp.int32),
            pltpu.VMEM((BLK,), jnp.int32),
            pltpu.VMEM((b // 16,), jnp.int32),
            pltpu.VMEM((b // 16,), jnp.int32),
            pltpu.VMEM((BLK, dh), jnp.float32),
            pltpu.VMEM((BLK, 16), jnp.float32),
            pltpu.VMEM((n1,), jnp.int32),
            pltpu.VMEM((b,), jnp.int32),
            pltpu.VMEM((chunk0 // 4,), jnp.int32),
            pltpu.VMEM((chunk0 // 4 + 2 * BLK,), jnp.int32),
            pltpu.SemaphoreType.DMA,
            pltpu.SemaphoreType.DMA,
            pltpu.SemaphoreType.DMA,
        ],
        compiler_params=pltpu.CompilerParams(use_tc_tiling_on_sc=False,
                                             needs_layout_passes=False),
    )
    x1n, s1, a2n, x1e, a2e = sc(nf0, nf1, edge_features,
                                from0x, to0x, eid0x, nodes1,
                                from1, to1_pos, eidx1, batch_pos,
                                z64, z16, zi)

    return pl.pallas_call(
        _tc_body,
        out_shape=jax.ShapeDtypeStruct((b, d), jnp.float32),
    )(s1, x1n, a2n, x1e, a2e, W1, W2)
